# baseline (speedup 1.0000x reference)
import functools

import jax
import jax.numpy as jnp
from jax import lax
from jax.experimental import pallas as pl
from jax.experimental.pallas import tpu as pltpu
from jax.experimental.pallas import tpu_sc as plsc

_NC = 2
_NS = 16
_NW = _NC * _NS
_CHUNK = 128


@functools.lru_cache(maxsize=None)
def _make_sc_gather(V, D, B):
    bpw = B // _NW
    n_chunks = bpw // _CHUNK
    mesh = plsc.VectorSubcoreMesh(core_axis_name="c", subcore_axis_name="s")

    @functools.partial(
        pl.kernel,
        mesh=mesh,
        out_type=jax.ShapeDtypeStruct((5 * B, 2 * D), jnp.float32),
        scratch_types=[
            pltpu.VMEM((bpw,), jnp.int32),
            pltpu.VMEM((bpw, D), jnp.float32),
            pltpu.SemaphoreType.DMA,
            pltpu.SemaphoreType.DMA,
        ],
        compiler_params=pltpu.CompilerParams(use_tc_tiling_on_sc=False),
    )
    def sc_gather(table_hbm, idx_hbm, out_hbm, idx_v, rows_v, gsem, wsem):
        wid = lax.axis_index("s") * _NC + lax.axis_index("c")
        base = wid * bpw
        pltpu.sync_copy(idx_hbm.at[pl.ds(base, bpw)], idx_v)
        gathers = []
        for c in range(n_chunks):
            gathers.append(
                pltpu.async_copy(
                    table_hbm.at[idx_v.at[pl.ds(c * _CHUNK, _CHUNK)]],
                    rows_v.at[pl.ds(c * _CHUNK, _CHUNK)],
                    gsem,
                )
            )
        writes = []
        for c in range(n_chunks):
            gathers[c].wait()
            writes.append(
                pltpu.async_copy(
                    rows_v.at[pl.ds(c * _CHUNK, _CHUNK)],
                    out_hbm.at[pl.ds(base + c * _CHUNK, _CHUNK), pl.ds(0, D)],
                    wsem,
                )
            )
        for w in writes:
            w.wait()

    return sc_gather


@functools.lru_cache(maxsize=None)
def _make_transpose(D, B, blk=2048):
    n = B // blk

    def body(in_hbm, out_ref, buf, sems):
        i = pl.program_id(0)

        @pl.when(i == 0)
        def _():
            for k in range(n):
                pltpu.make_async_copy(
                    in_hbm.at[pl.ds(k * blk, blk), pl.ds(0, 2 * D)],
                    buf.at[k],
                    sems.at[k],
                ).start()

        pltpu.make_async_copy(
            in_hbm.at[pl.ds(i * blk, blk), pl.ds(0, 2 * D)], buf.at[i], sems.at[i]
        ).wait()
        out_ref[...] = buf[i].T[:D, :]

    def run(x):
        return pl.pallas_call(
            body,
            out_shape=jax.ShapeDtypeStruct((D, B), jnp.float32),
            grid=(n,),
            in_specs=[pl.BlockSpec(memory_space=pl.ANY)],
            out_specs=pl.BlockSpec((D, blk), lambda i: (0, i)),
            scratch_shapes=[
                pltpu.VMEM((n, blk, 2 * D), jnp.float32),
                pltpu.SemaphoreType.DMA((n,)),
            ],
        )(x)

    return run


def kernel(speaker, embedding_table):
    idx = speaker.astype(jnp.int32)
    (B,) = idx.shape
    V, D = embedding_table.shape
    staged = _make_sc_gather(V, D, B)(embedding_table, idx)
    out_t = _make_transpose(D, B)(staged)
    return out_t.T
